# baseline (speedup 1.0000x reference)
import functools

import jax
import jax.numpy as jnp
from jax import lax
from jax.experimental import pallas as pl
from jax.experimental.pallas import tpu as pltpu
from jax.experimental.pallas import tpu_sc as plsc

EMB = 64
HID = 256
BATCH = 16384
NU = 1000000
NI = 100000

NC = 2
NS = 16
NW = NC * NS
L = 16
SLABW = 512
NBUCK = 2

UCPW = (NU // SLABW + 1 + NW - 1) // NW
ICPW = (NI // SLABW + 1 + NW - 1) // NW


def _phase(wid, ids_hbm, tab_hbm, tail_hbm, out_hbm, ncols, cpw,
           ids_v, mids_v, mpos_v, slab_v, tail_v, rowbuf_v, posb_v, sem):
    nch_total = (ncols + SLABW - 1) // SLABW
    nfull = ncols // SLABW
    tailw = ncols - nfull * SLABW
    lanes = lax.iota(jnp.int32, L)
    c0 = wid * cpw
    lo = c0 * SLABW
    hi = lo + jnp.clip(nch_total - c0, 0, cpw) * SLABW

    pltpu.sync_copy(ids_hbm, ids_v)

    def scanv(v, ptr):
        vec = ids_v[pl.ds(v * L, L)]
        inm = (vec >= lo) & (vec < hi)
        cnt = plsc.all_reduce_population_count(inm)[0]
        plsc.store_compressed(mids_v.at[pl.ds(ptr, L)], vec, mask=inm)
        plsc.store_compressed(mpos_v.at[pl.ds(ptr, L)], v * L + lanes,
                              mask=inm)
        return ptr + cnt

    m_cnt = lax.fori_loop(0, BATCH // L, scanv, 0)
    mids_v[pl.ds(m_cnt, L)] = jnp.full((L,), 1 << 30, jnp.int32)
    ngrp = (m_cnt + L - 1) // L

    def process_chunk(clo, width, scnt, sref):
        def group(g, sc):
            mvec = mids_v[pl.ds(g * L, L)]
            inm = ((mvec >= clo) & (mvec < clo + width)).astype(jnp.int32)
            cum = jnp.cumsum(inm)
            cnt = cum[L - 1]

            @pl.when(cnt > 0)
            def _():
                pvec = mpos_v[pl.ds(g * L, L)]
                rank = cum - 1
                slotv = sc + rank
                plsc.store_scatter(
                    posb_v, [(slotv // L) % NBUCK, slotv % L], pvec,
                    mask=inm > 0)
                colv = mvec - clo
                for m in range(L):
                    @pl.when(inm[m] > 0)
                    def _():
                        slot = (sc + rank[m]) % (NBUCK * L)
                        colsp = jnp.full((L,), colv[m], jnp.int32)
                        for q in range(EMB // L):
                            rowbuf_v[slot, pl.ds(q * L, L)] = (
                                plsc.load_gather(sref,
                                                 [q * L + lanes, colsp]))

                @pl.when((sc % L) + cnt >= L)
                def _():
                    b = (sc // L) % NBUCK
                    pltpu.async_copy(rowbuf_v.at[pl.ds(b * L, L)],
                                     out_hbm.at[posb_v.at[b]], sem).wait()

            return sc + cnt

        return lax.fori_loop(0, ngrp, group, scnt)

    n_full_mine = jnp.clip(nfull - c0, 0, cpw)

    def chunk_loop(c, sc):
        pltpu.sync_copy(tab_hbm.at[:, pl.ds((c0 + c) * SLABW, SLABW)], slab_v)
        return process_chunk((c0 + c) * SLABW, SLABW, sc, slab_v)

    scnt = lax.fori_loop(0, n_full_mine, chunk_loop, 0)
    if tailw:
        pltpu.sync_copy(tail_hbm, tail_v)
        scnt = process_chunk(nfull * SLABW, tailw, scnt, tail_v)

    rfill = scnt % L

    @pl.when(rfill > 0)
    def _():
        b = (scnt // L) % NBUCK
        base = b * L
        pv = posb_v[b, pl.ds(0, L)]
        posb_v[b, pl.ds(0, L)] = jnp.where(lanes < rfill, pv, pv[0])
        for m in range(1, L):
            @pl.when(jnp.int32(m) >= rfill)
            def _():
                for q in range(EMB // L):
                    rowbuf_v[base + m, pl.ds(q * L, L)] = (
                        rowbuf_v[base, pl.ds(q * L, L)])
        pltpu.async_copy(rowbuf_v.at[pl.ds(base, L)],
                         out_hbm.at[posb_v.at[b]], sem).wait()


TAILU = NU % SLABW
TAILI = NI % SLABW


def _scan_body(uids_hbm, iids_hbm, uT_hbm, iT_hbm, utail_hbm, itail_hbm,
               uout_hbm, iout_hbm,
               ids_v, mids_v, mpos_v, slab_v, utail_v, itail_v,
               rowbuf_v, posb_v, sem):
    wid = lax.axis_index("s") * NC + lax.axis_index("c")
    _phase(wid, uids_hbm, uT_hbm, utail_hbm, uout_hbm, NU, UCPW,
           ids_v, mids_v, mpos_v, slab_v, utail_v, rowbuf_v, posb_v, sem)
    _phase(wid, iids_hbm, iT_hbm, itail_hbm, iout_hbm, NI, ICPW,
           ids_v, mids_v, mpos_v, slab_v, itail_v, rowbuf_v, posb_v, sem)


@jax.jit
def _scan_gather(user_ids, item_ids, u_t, i_t, u_tail, i_tail):
    mesh = plsc.VectorSubcoreMesh(core_axis_name="c", subcore_axis_name="s")
    fn = functools.partial(
        pl.kernel,
        mesh=mesh,
        compiler_params=pltpu.CompilerParams(needs_layout_passes=False),
        out_type=[
            jax.ShapeDtypeStruct((BATCH, 2 * EMB), jnp.float32),
            jax.ShapeDtypeStruct((BATCH, 2 * EMB), jnp.float32),
        ],
        scratch_types=[
            pltpu.VMEM((BATCH,), jnp.int32),
            pltpu.VMEM((BATCH + L,), jnp.int32),
            pltpu.VMEM((BATCH + L,), jnp.int32),
            pltpu.VMEM((EMB, SLABW), jnp.float32),
            pltpu.VMEM((EMB, TAILU), jnp.float32),
            pltpu.VMEM((EMB, TAILI), jnp.float32),
            pltpu.VMEM((NBUCK * L, 2 * EMB), jnp.float32),
            pltpu.VMEM((NBUCK, L), jnp.int32),
            pltpu.SemaphoreType.DMA,
        ],
    )(_scan_body)
    return fn(user_ids, item_ids, u_t, i_t, u_tail, i_tail)


BS = 2048


def _mlp_body(u_ref, i_ref, w1u_ref, w1i_ref, b1_ref, w2t_ref, b2_ref,
              out_ref):
    x = jnp.dot(u_ref[:, 0:EMB], w1u_ref[...],
                preferred_element_type=jnp.float32)
    x = x + jnp.dot(i_ref[:, 0:EMB], w1i_ref[...],
                    preferred_element_type=jnp.float32)
    x = jnp.maximum(x + b1_ref[...], 0.0)
    y = jnp.sum(x * w2t_ref[...], axis=1, keepdims=True)
    out_ref[...] = y + b2_ref[...]


@jax.jit
def _mlp(u128, i128, w1u, w1i, b1, w2t, b2):
    grid = (BATCH // BS,)
    return pl.pallas_call(
        _mlp_body,
        grid=grid,
        in_specs=[
            pl.BlockSpec((BS, 2 * EMB), lambda g: (g, 0)),
            pl.BlockSpec((BS, 2 * EMB), lambda g: (g, 0)),
            pl.BlockSpec((EMB, HID), lambda g: (0, 0)),
            pl.BlockSpec((EMB, HID), lambda g: (0, 0)),
            pl.BlockSpec((1, HID), lambda g: (0, 0)),
            pl.BlockSpec((1, HID), lambda g: (0, 0)),
            pl.BlockSpec((1, 1), lambda g: (0, 0)),
        ],
        out_specs=pl.BlockSpec((BS, 1), lambda g: (g, 0)),
        out_shape=jax.ShapeDtypeStruct((BATCH, 1), jnp.float32),
    )(u128, i128, w1u, w1i, b1, w2t, b2)


def kernel(user_ids, item_ids, user_emb, item_emb, W1, b1, W2, b2):
    uids = user_ids.astype(jnp.int32)
    iids = item_ids.astype(jnp.int32)
    u_t = user_emb.T
    i_t = item_emb.T
    u_tail = lax.slice(u_t, (0, NU - TAILU), (EMB, NU))
    i_tail = lax.slice(i_t, (0, NI - TAILI), (EMB, NI))
    u128, i128 = _scan_gather(uids, iids, u_t, i_t, u_tail, i_tail)
    return _mlp(u128, i128, W1[:EMB], W1[EMB:], b1.reshape(1, HID),
                W2.reshape(1, HID), b2.reshape(1, 1))


# speedup vs baseline: 1.1921x; 1.1921x over previous
import functools

import jax
import jax.numpy as jnp
from jax import lax
from jax.experimental import pallas as pl
from jax.experimental.pallas import tpu as pltpu
from jax.experimental.pallas import tpu_sc as plsc

EMB = 64
HID = 256
BATCH = 16384
NU = 1000000
NI = 100000

NC = 2
NS = 16
NW = NC * NS
L = 16
SLABW = 384
NBUCK = 2

UCPW = (NU // SLABW + 1 + NW - 1) // NW
ICPW = (NI // SLABW + 1 + NW - 1) // NW


def _phase(wid, ids_hbm, tab_hbm, tail_hbm, out_hbm, ncols, cpw,
           ids_v, mids_v, mpos_v, slab_v, tail_v, rowbuf_v, posb_v, sem,
           semslab):
    nch_total = (ncols + SLABW - 1) // SLABW
    nfull = ncols // SLABW
    tailw = ncols - nfull * SLABW
    lanes = lax.iota(jnp.int32, L)
    c0 = wid * cpw
    lo = c0 * SLABW
    hi = lo + jnp.clip(nch_total - c0, 0, cpw) * SLABW

    pltpu.sync_copy(ids_hbm, ids_v)

    def scanv(v, ptr):
        vec = ids_v[pl.ds(v * L, L)]
        inm = (vec >= lo) & (vec < hi)
        cnt = plsc.all_reduce_population_count(inm)[0]
        plsc.store_compressed(mids_v.at[pl.ds(ptr, L)], vec, mask=inm)
        plsc.store_compressed(mpos_v.at[pl.ds(ptr, L)], v * L + lanes,
                              mask=inm)
        return ptr + cnt

    m_cnt = lax.fori_loop(0, BATCH // L, scanv, 0)
    mids_v[pl.ds(m_cnt, L)] = jnp.full((L,), 1 << 30, jnp.int32)
    ngrp = (m_cnt + L - 1) // L

    def process_chunk(clo, width, scnt, sref):
        def group(g, sc):
            mvec = mids_v[pl.ds(g * L, L)]
            inm = ((mvec >= clo) & (mvec < clo + width)).astype(jnp.int32)
            cum = jnp.cumsum(inm)
            cnt = cum[L - 1]

            @pl.when(cnt > 0)
            def _():
                pvec = mpos_v[pl.ds(g * L, L)]
                rank = cum - 1
                slotv = sc + rank
                plsc.store_scatter(
                    posb_v, [(slotv // L) % NBUCK, slotv % L], pvec,
                    mask=inm > 0)
                colv = mvec - clo
                for m in range(L):
                    @pl.when(inm[m] > 0)
                    def _():
                        slot = (sc + rank[m]) % (NBUCK * L)
                        colsp = jnp.full((L,), colv[m], jnp.int32)
                        for q in range(EMB // L):
                            rowbuf_v[slot, pl.ds(q * L, L)] = (
                                plsc.load_gather(sref,
                                                 [q * L + lanes, colsp]))

                @pl.when((sc % L) + cnt >= L)
                def _():
                    b = (sc // L) % NBUCK
                    pltpu.async_copy(rowbuf_v.at[pl.ds(b * L, L)],
                                     out_hbm.at[posb_v.at[b]], sem).wait()

            return sc + cnt

        return lax.fori_loop(0, ngrp, group, scnt)

    n_full_mine = jnp.clip(nfull - c0, 0, cpw)

    def slab_copy(c, b):
        return pltpu.make_async_copy(
            tab_hbm.at[:, pl.ds((c0 + c) * SLABW, SLABW)], slab_v.at[b],
            semslab)

    def chunk_loop(c, sc):
        @pl.when(c + 1 < n_full_mine)
        def _():
            slab_copy(c + 1, (c + 1) % 2).start()
        slab_copy(c, c % 2).wait()
        return process_chunk((c0 + c) * SLABW, SLABW, sc, slab_v.at[c % 2])

    @pl.when(n_full_mine > 0)
    def _():
        slab_copy(0, 0).start()

    scnt = lax.fori_loop(0, n_full_mine, chunk_loop, 0)
    if tailw:
        pltpu.sync_copy(tail_hbm, tail_v)
        scnt = process_chunk(nfull * SLABW, tailw, scnt, tail_v)

    rfill = scnt % L

    @pl.when(rfill > 0)
    def _():
        b = (scnt // L) % NBUCK
        base = b * L
        pv = posb_v[b, pl.ds(0, L)]
        posb_v[b, pl.ds(0, L)] = jnp.where(lanes < rfill, pv, pv[0])
        for m in range(1, L):
            @pl.when(jnp.int32(m) >= rfill)
            def _():
                for q in range(EMB // L):
                    rowbuf_v[base + m, pl.ds(q * L, L)] = (
                        rowbuf_v[base, pl.ds(q * L, L)])
        pltpu.async_copy(rowbuf_v.at[pl.ds(base, L)],
                         out_hbm.at[posb_v.at[b]], sem).wait()


TAILU = NU % SLABW
TAILI = NI % SLABW


def _scan_body(uids_hbm, iids_hbm, uT_hbm, iT_hbm, utail_hbm, itail_hbm,
               uout_hbm, iout_hbm,
               ids_v, mids_v, mpos_v, slab_v, utail_v, itail_v,
               rowbuf_v, posb_v, sem, semslab):
    wid = lax.axis_index("s") * NC + lax.axis_index("c")
    _phase(wid, uids_hbm, uT_hbm, utail_hbm, uout_hbm, NU, UCPW,
           ids_v, mids_v, mpos_v, slab_v, utail_v, rowbuf_v, posb_v, sem,
           semslab)
    _phase(wid, iids_hbm, iT_hbm, itail_hbm, iout_hbm, NI, ICPW,
           ids_v, mids_v, mpos_v, slab_v, itail_v, rowbuf_v, posb_v, sem,
           semslab)


@jax.jit
def _scan_gather(user_ids, item_ids, u_t, i_t, u_tail, i_tail):
    mesh = plsc.VectorSubcoreMesh(core_axis_name="c", subcore_axis_name="s")
    fn = functools.partial(
        pl.kernel,
        mesh=mesh,
        compiler_params=pltpu.CompilerParams(needs_layout_passes=False),
        out_type=[
            jax.ShapeDtypeStruct((BATCH, 2 * EMB), jnp.float32),
            jax.ShapeDtypeStruct((BATCH, 2 * EMB), jnp.float32),
        ],
        scratch_types=[
            pltpu.VMEM((BATCH,), jnp.int32),
            pltpu.VMEM((BATCH + L,), jnp.int32),
            pltpu.VMEM((BATCH + L,), jnp.int32),
            pltpu.VMEM((2, EMB, SLABW), jnp.float32),
            pltpu.VMEM((EMB, TAILU), jnp.float32),
            pltpu.VMEM((EMB, TAILI), jnp.float32),
            pltpu.VMEM((NBUCK * L, 2 * EMB), jnp.float32),
            pltpu.VMEM((NBUCK, L), jnp.int32),
            pltpu.SemaphoreType.DMA,
            pltpu.SemaphoreType.DMA,
        ],
    )(_scan_body)
    return fn(user_ids, item_ids, u_t, i_t, u_tail, i_tail)


BS = 2048


def _mlp_body(u_ref, i_ref, w1u_ref, w1i_ref, b1_ref, w2t_ref, b2_ref,
              out_ref):
    x = jnp.dot(u_ref[:, 0:EMB], w1u_ref[...],
                preferred_element_type=jnp.float32)
    x = x + jnp.dot(i_ref[:, 0:EMB], w1i_ref[...],
                    preferred_element_type=jnp.float32)
    x = jnp.maximum(x + b1_ref[...], 0.0)
    y = jnp.sum(x * w2t_ref[...], axis=1, keepdims=True)
    out_ref[...] = y + b2_ref[...]


@jax.jit
def _mlp(u128, i128, w1u, w1i, b1, w2t, b2):
    grid = (BATCH // BS,)
    return pl.pallas_call(
        _mlp_body,
        grid=grid,
        in_specs=[
            pl.BlockSpec((BS, 2 * EMB), lambda g: (g, 0)),
            pl.BlockSpec((BS, 2 * EMB), lambda g: (g, 0)),
            pl.BlockSpec((EMB, HID), lambda g: (0, 0)),
            pl.BlockSpec((EMB, HID), lambda g: (0, 0)),
            pl.BlockSpec((1, HID), lambda g: (0, 0)),
            pl.BlockSpec((1, HID), lambda g: (0, 0)),
            pl.BlockSpec((1, 1), lambda g: (0, 0)),
        ],
        out_specs=pl.BlockSpec((BS, 1), lambda g: (g, 0)),
        out_shape=jax.ShapeDtypeStruct((BATCH, 1), jnp.float32),
    )(u128, i128, w1u, w1i, b1, w2t, b2)


def kernel(user_ids, item_ids, user_emb, item_emb, W1, b1, W2, b2):
    uids = user_ids.astype(jnp.int32)
    iids = item_ids.astype(jnp.int32)
    u_t = user_emb.T
    i_t = item_emb.T
    u_tail = lax.slice(u_t, (0, NU - TAILU), (EMB, NU))
    i_tail = lax.slice(i_t, (0, NI - TAILI), (EMB, NI))
    u128, i128 = _scan_gather(uids, iids, u_t, i_t, u_tail, i_tail)
    return _mlp(u128, i128, W1[:EMB], W1[EMB:], b1.reshape(1, HID),
                W2.reshape(1, HID), b2.reshape(1, 1))


# speedup vs baseline: 1.9811x; 1.6619x over previous
import functools

import jax
import jax.numpy as jnp
from jax import lax
from jax.experimental import pallas as pl
from jax.experimental.pallas import tpu as pltpu
from jax.experimental.pallas import tpu_sc as plsc

EMB = 64
HID = 256
BATCH = 16384

NC = 2
NS = 16
NW = NC * NS
CHUNK = 128
B_PER_W = BATCH // NW
K = B_PER_W // CHUNK
L = 16


HU = 507904
HI = 57344


def _gather_body(uids_hbm, iids_hbm, u2_hbm, i2_hbm, comb_hbm,
                 uids_v, iids_v, uidx_v, iidx_v, ustage_v, istage_v, comb_v,
                 semu, semi):
    wid = lax.axis_index("s") * NC + lax.axis_index("c")
    base = wid * B_PER_W
    for j in range(K):
        pltpu.sync_copy(uids_hbm.at[pl.ds(base + j * CHUNK, CHUNK)], uids_v.at[j])
        pltpu.sync_copy(iids_hbm.at[pl.ds(base + j * CHUNK, CHUNK)], iids_v.at[j])
    for j in range(K):
        for g in range(CHUNK // L):
            uvec = uids_v[j, pl.ds(g * L, L)]
            ivec = iids_v[j, pl.ds(g * L, L)]
            uidx_v[j, pl.ds(g * L, L)] = uvec - jnp.where(uvec >= HU, HU, 0)
            iidx_v[j, pl.ds(g * L, L)] = ivec - jnp.where(ivec >= HI, HI, 0)

    def fire(j):
        b = j % 2
        pltpu.async_copy(u2_hbm.at[uidx_v.at[j]], ustage_v.at[b], semu)
        pltpu.async_copy(i2_hbm.at[iidx_v.at[j]], istage_v.at[b], semi)

    def wait(j):
        b = j % 2
        pltpu.make_async_copy(u2_hbm.at[uidx_v.at[j]], ustage_v.at[b], semu).wait()
        pltpu.make_async_copy(i2_hbm.at[iidx_v.at[j]], istage_v.at[b], semi).wait()

    def extract(j):
        b = j % 2

        def group(t, _):
            uvec = uids_v[j, pl.ds(t * L, L)]
            ivec = iids_v[j, pl.ds(t * L, L)]
            for m in range(L):
                r = t * L + m
                uoff = jnp.where(uvec[m] >= HU, EMB, 0)
                ioff = jnp.where(ivec[m] >= HI, EMB, 0)
                for q in range(EMB // L):
                    comb_v[r, pl.ds(q * L, L)] = (
                        ustage_v[b, r, pl.ds(uoff + q * L, L)])
                    comb_v[r, pl.ds(EMB + q * L, L)] = (
                        istage_v[b, r, pl.ds(ioff + q * L, L)])
            return 0

        lax.fori_loop(0, CHUNK // L, group, 0)

    fire(0)
    for j in range(K):
        if j + 1 < K:
            fire(j + 1)
        wait(j)
        extract(j)
        pltpu.sync_copy(comb_v, comb_hbm.at[pl.ds(base + j * CHUNK, CHUNK)])


@jax.jit
def _gather(user_ids, item_ids, u2, i2):
    mesh = plsc.VectorSubcoreMesh(core_axis_name="c", subcore_axis_name="s")
    fn = functools.partial(
        pl.kernel,
        mesh=mesh,
        out_type=jax.ShapeDtypeStruct((BATCH, 2 * EMB), jnp.float32),
        scratch_types=[
            pltpu.VMEM((K, CHUNK), jnp.int32),
            pltpu.VMEM((K, CHUNK), jnp.int32),
            pltpu.VMEM((K, CHUNK), jnp.int32),
            pltpu.VMEM((K, CHUNK), jnp.int32),
            pltpu.VMEM((2, CHUNK, 2 * EMB), jnp.float32),
            pltpu.VMEM((2, CHUNK, 2 * EMB), jnp.float32),
            pltpu.VMEM((CHUNK, 2 * EMB), jnp.float32),
            pltpu.SemaphoreType.DMA,
            pltpu.SemaphoreType.DMA,
        ],
    )(_gather_body)
    return fn(user_ids, item_ids, u2, i2)


def _tpose_body(a_ref, b_ref, out_ref):
    out_ref[:, 0:EMB] = jnp.swapaxes(a_ref[...], 0, 1)
    out_ref[:, EMB:2 * EMB] = jnp.swapaxes(b_ref[...], 0, 1)


TWU = 16384
TWI = 8192


def _tpose(tab_t, h, tw):
    nblk = h // tw
    bmax = tab_t.shape[1] // tw
    return pl.pallas_call(
        _tpose_body,
        grid=(nblk,),
        in_specs=[
            pl.BlockSpec((EMB, tw), lambda g: (0, g)),
            pl.BlockSpec((EMB, tw), lambda g: (0, jnp.minimum(g + nblk, bmax))),
        ],
        out_specs=pl.BlockSpec((tw, 2 * EMB), lambda g: (g, 0)),
        out_shape=jax.ShapeDtypeStruct((h, 2 * EMB), jnp.float32),
    )(tab_t, tab_t)


BS = 2048


def _mlp_body(c_ref, w1_ref, b1_ref, w2t_ref, b2_ref, out_ref):
    x = jnp.dot(c_ref[...], w1_ref[...], preferred_element_type=jnp.float32)
    x = jnp.maximum(x + b1_ref[...], 0.0)
    y = jnp.sum(x * w2t_ref[...], axis=1, keepdims=True)
    out_ref[...] = y + b2_ref[...]


@jax.jit
def _mlp(comb, w1, b1, w2t, b2):
    grid = (BATCH // BS,)
    return pl.pallas_call(
        _mlp_body,
        grid=grid,
        in_specs=[
            pl.BlockSpec((BS, 2 * EMB), lambda g: (g, 0)),
            pl.BlockSpec((2 * EMB, HID), lambda g: (0, 0)),
            pl.BlockSpec((1, HID), lambda g: (0, 0)),
            pl.BlockSpec((1, HID), lambda g: (0, 0)),
            pl.BlockSpec((1, 1), lambda g: (0, 0)),
        ],
        out_specs=pl.BlockSpec((BS, 1), lambda g: (g, 0)),
        out_shape=jax.ShapeDtypeStruct((BATCH, 1), jnp.float32),
    )(comb, w1, b1, w2t, b2)


def kernel(user_ids, item_ids, user_emb, item_emb, W1, b1, W2, b2):
    uids = user_ids.astype(jnp.int32)
    iids = item_ids.astype(jnp.int32)
    u2 = _tpose(user_emb.T, HU, TWU)
    i2 = _tpose(item_emb.T, HI, TWI)
    comb = _gather(uids, iids, u2, i2)
    return _mlp(comb, W1, b1.reshape(1, HID), W2.reshape(1, HID),
                b2.reshape(1, 1))
